# baseline (speedup 1.0000x reference)
import jax
import jax.numpy as jnp
from jax.experimental import pallas as pl
from jax.experimental.pallas import tpu as pltpu

B, N, D = 8, 2048, 32
BLK = 512
S = N // BLK
NSLAB = B * S
NBUF = 4


def _gcn_manual_kernel(text_ref, adj_hbm, w_ref, b_ref, out_ref, buf, hidden_ref, sems):
    i = pl.program_id(0)

    @pl.when(i == 0)
    def _():
        for k in range(NBUF):
            bk = k // S
            rk = (k % S) * BLK
            pltpu.make_async_copy(
                adj_hbm.at[bk, pl.ds(rk, BLK), :], buf.at[k], sems.at[k]
            ).start()

    @pl.when(i % S == 0)
    def _():
        hidden_ref[:, :D] = jnp.dot(
            text_ref[0], w_ref[...], preferred_element_type=jnp.float32
        )
        hidden_ref[:, D:] = jnp.ones((N, D), jnp.float32)

    slot = i % NBUF
    pltpu.make_async_copy(
        adj_hbm.at[0, pl.ds(0, BLK), :], buf.at[slot], sems.at[slot]
    ).wait()

    a = buf[slot]
    acc = jnp.dot(a, hidden_ref[...], preferred_element_type=jnp.float32)
    denom = acc[:, D : D + 1] + 1.0
    out_ref[0] = acc[:, :D] / denom + b_ref[...]

    nxt = i + NBUF

    @pl.when(nxt < NSLAB)
    def _():
        bn = nxt // S
        rn = (nxt % S) * BLK
        pltpu.make_async_copy(
            adj_hbm.at[bn, pl.ds(rn, BLK), :], buf.at[slot], sems.at[slot]
        ).start()


def kernel(text, adj, W, b):
    b2d = b.reshape(1, D)
    grid = (NSLAB,)
    return pl.pallas_call(
        _gcn_manual_kernel,
        grid=grid,
        in_specs=[
            pl.BlockSpec((1, N, D), lambda i: (i // S, 0, 0)),
            pl.BlockSpec(memory_space=pltpu.MemorySpace.HBM),
            pl.BlockSpec((D, D), lambda i: (0, 0)),
            pl.BlockSpec((1, D), lambda i: (0, 0)),
        ],
        out_specs=pl.BlockSpec((1, BLK, D), lambda i: (i // S, i % S, 0)),
        out_shape=jax.ShapeDtypeStruct((B, N, D), jnp.float32),
        scratch_shapes=[
            pltpu.VMEM((NBUF, BLK, N), jnp.float32),
            pltpu.VMEM((N, 2 * D), jnp.float32),
            pltpu.SemaphoreType.DMA((NBUF,)),
        ],
        compiler_params=pltpu.CompilerParams(
            dimension_semantics=("arbitrary",),
        ),
    )(text, adj, W, b2d)


# speedup vs baseline: 1.0980x; 1.0980x over previous
import jax
import jax.numpy as jnp
from jax.experimental import pallas as pl
from jax.experimental.pallas import tpu as pltpu

B, N, D = 8, 2048, 32


def _gcn_fused_kernel(text_ref, adj_ref, w_ref, b_ref, out_ref, hidden_ref):
    bi = pl.program_id(0)
    hidden_ref[:, :D] = jnp.dot(
        text_ref[bi], w_ref[...], preferred_element_type=jnp.float32
    )
    hidden_ref[:, D:] = jnp.ones((N, D), jnp.float32)

    a = adj_ref[0]
    acc = jnp.dot(a, hidden_ref[...], preferred_element_type=jnp.float32)
    denom = acc[:, D : D + 1] + 1.0
    out_ref[0] = acc[:, :D] / denom + b_ref[...]


def kernel(text, adj, W, b):
    b2d = b.reshape(1, D)
    return pl.pallas_call(
        _gcn_fused_kernel,
        grid=(B,),
        in_specs=[
            pl.BlockSpec((B, N, D), lambda bi: (0, 0, 0)),
            pl.BlockSpec((1, N, N), lambda bi: (bi, 0, 0)),
            pl.BlockSpec((D, D), lambda bi: (0, 0)),
            pl.BlockSpec((1, D), lambda bi: (0, 0)),
        ],
        out_specs=pl.BlockSpec((1, N, D), lambda bi: (bi, 0, 0)),
        out_shape=jax.ShapeDtypeStruct((B, N, D), jnp.float32),
        scratch_shapes=[pltpu.VMEM((N, 2 * D), jnp.float32)],
        compiler_params=pltpu.CompilerParams(
            dimension_semantics=("arbitrary",),
        ),
    )(text, adj, W, b2d)


# speedup vs baseline: 1.1626x; 1.0589x over previous
import jax
import jax.numpy as jnp
from jax.experimental import pallas as pl
from jax.experimental.pallas import tpu as pltpu

B, N, D = 8, 2048, 32


def _gcn_fused_kernel(text_ref, adj_ref, w_ref, b_ref, out_ref, hidden_ref):
    bi = pl.program_id(0)

    @pl.when(bi == 0)
    def _():
        hidden_ref[0, :, :D] = jnp.dot(
            text_ref[0], w_ref[...], preferred_element_type=jnp.float32
        )
        hidden_ref[0, :, D:] = jnp.ones((N, D), jnp.float32)

    a = adj_ref[0]
    acc = jnp.dot(a, hidden_ref[bi % 2], preferred_element_type=jnp.float32)
    denom = acc[:, D : D + 1] + 1.0
    out_ref[0] = acc[:, :D] / denom + b_ref[...]

    @pl.when(bi + 1 < B)
    def _():
        nxt = (bi + 1) % 2
        hidden_ref[nxt, :, :D] = jnp.dot(
            text_ref[bi + 1], w_ref[...], preferred_element_type=jnp.float32
        )
        hidden_ref[nxt, :, D:] = jnp.ones((N, D), jnp.float32)


def kernel(text, adj, W, b):
    b2d = b.reshape(1, D)
    return pl.pallas_call(
        _gcn_fused_kernel,
        grid=(B,),
        in_specs=[
            pl.BlockSpec((B, N, D), lambda bi: (0, 0, 0)),
            pl.BlockSpec((1, N, N), lambda bi: (bi, 0, 0)),
            pl.BlockSpec((D, D), lambda bi: (0, 0)),
            pl.BlockSpec((1, D), lambda bi: (0, 0)),
        ],
        out_specs=pl.BlockSpec((1, N, D), lambda bi: (bi, 0, 0)),
        out_shape=jax.ShapeDtypeStruct((B, N, D), jnp.float32),
        scratch_shapes=[pltpu.VMEM((2, N, 2 * D), jnp.float32)],
        compiler_params=pltpu.CompilerParams(
            dimension_semantics=("arbitrary",),
        ),
    )(text, adj, W, b2d)
